# baseline (speedup 1.0000x reference)
import functools

import jax
import jax.numpy as jnp
from jax import lax
from jax.experimental import pallas as pl
from jax.experimental.pallas import tpu as pltpu
from jax.experimental.pallas import tpu_sc as plsc

B, L = 1024, 200
N = B * L
D_W, D_P, D_T = 128, 32, 16
D_PT = D_P + D_T
D_OUT = D_W + D_PT
N_POS, N_TYPES = 50, 30

NC, NS = 2, 16
NW = NC * NS
ROWS_PER_W = N // NW
CHUNK = 128
N_CHUNKS = ROWS_PER_W // CHUNK


def _make_sc_kernel():
    mesh = plsc.VectorSubcoreMesh(core_axis_name="c", subcore_axis_name="s")

    @functools.partial(
        pl.kernel,
        mesh=mesh,
        out_type=jax.ShapeDtypeStruct((N, D_OUT), jnp.float32),
        scratch_types=[
            pltpu.VMEM((CHUNK,), jnp.int32),
            pltpu.VMEM((CHUNK,), jnp.int32),
            pltpu.VMEM((CHUNK,), jnp.int32),
            pltpu.VMEM((CHUNK,), jnp.int32),
            pltpu.VMEM((CHUNK, D_W), jnp.float32),
            pltpu.VMEM((CHUNK, D_W), jnp.float32),
            pltpu.VMEM((CHUNK, D_OUT), jnp.float32),
            pltpu.VMEM((CHUNK, D_OUT), jnp.float32),
            pltpu.SemaphoreType.DMA,
            pltpu.SemaphoreType.DMA,
            pltpu.SemaphoreType.DMA,
            pltpu.SemaphoreType.DMA,
        ],
    )
    def k(tok_hbm, pt_hbm, wt_hbm, ptf_hbm, out_hbm,
          tok_i0, tok_i1, pt_i0, pt_i1, bpt0, bpt1, comb0, comb1,
          g_sem0, g_sem1, i_sem0, i_sem1):
        wid = lax.axis_index("s") * NC + lax.axis_index("c")
        row0 = wid * ROWS_PER_W

        set0 = (tok_i0, pt_i0, bpt0, comb0, g_sem0, i_sem0)
        set1 = (tok_i1, pt_i1, bpt1, comb1, g_sem1, i_sem1)

        def start_idx(c, s):
            tok_i, pt_i, _, _, _, i_sem = s
            b = row0 + c * CHUNK
            pltpu.async_copy(tok_hbm.at[pl.ds(b, CHUNK)], tok_i, i_sem)
            pltpu.async_copy(pt_hbm.at[pl.ds(b, CHUNK)], pt_i, i_sem)

        def wait_idx(s):
            tok_i, pt_i, _, _, _, i_sem = s
            pltpu.make_async_copy(
                tok_hbm.at[pl.ds(0, CHUNK)], tok_i, i_sem).wait()
            pltpu.make_async_copy(
                pt_hbm.at[pl.ds(0, CHUNK)], pt_i, i_sem).wait()

        def issue_gathers(s):
            tok_i, pt_i, bpt, comb, g_sem, _ = s
            pltpu.async_copy(
                wt_hbm.at[tok_i], comb.at[:, pl.ds(0, D_W)], g_sem)
            pltpu.async_copy(ptf_hbm.at[pt_i], bpt, g_sem)

        def wait_gathers(s):
            _, _, bpt, comb, g_sem, _ = s
            pltpu.make_async_copy(
                wt_hbm.at[pl.ds(0, CHUNK)],
                comb.at[:, pl.ds(0, D_W)], g_sem).wait()
            pltpu.make_async_copy(
                ptf_hbm.at[pl.ds(0, CHUNK)], bpt, g_sem).wait()

        def phase(c, cur, nxt, do_prefetch_idx, do_issue_next):
            _, _, bpt, comb, _, _ = cur
            if do_issue_next:
                wait_idx(nxt)
                issue_gathers(nxt)
            wait_gathers(cur)

            def stitch(r, carry):
                row_src = bpt.at[r]
                row_dst = comb.at[r]
                row_dst[pl.ds(D_W, 16)] = row_src[pl.ds(0, 16)]
                row_dst[pl.ds(D_W + 16, 16)] = row_src[pl.ds(16, 16)]
                row_dst[pl.ds(D_W + 32, 16)] = row_src[pl.ds(32, 16)]
                return carry

            lax.fori_loop(0, CHUNK, stitch, 0)
            pltpu.sync_copy(comb, out_hbm.at[pl.ds(row0 + c * CHUNK, CHUNK)])
            if do_prefetch_idx:
                start_idx(c + 2, cur)

        start_idx(0, set0)
        start_idx(1, set1)
        wait_idx(set0)
        issue_gathers(set0)

        def body(j, carry):
            c = 2 * j
            phase(c, set0, set1, True, True)
            phase(c + 1, set1, set0, True, True)
            return carry

        lax.fori_loop(0, (N_CHUNKS - 2) // 2, body, 0)
        phase(N_CHUNKS - 2, set0, set1, False, True)
        phase(N_CHUNKS - 1, set1, set0, False, False)

    return k


_sc_kernel = _make_sc_kernel()

TC_BB = 8


def _tc_reshape(x):
    def body(x_ref, o_ref):
        o_ref[...] = x_ref[...].reshape(TC_BB, L, D_OUT)

    return pl.pallas_call(
        body,
        out_shape=jax.ShapeDtypeStruct((B, L, D_OUT), jnp.float32),
        grid=(B // TC_BB,),
        in_specs=[pl.BlockSpec((TC_BB * L, D_OUT), lambda i: (i, 0))],
        out_specs=pl.BlockSpec((TC_BB, L, D_OUT), lambda i: (i, 0, 0)),
    )(x)


def kernel(tokens, part_of_speeches, types, word_table, pos_table, type_table):
    tok = tokens.reshape(N).astype(jnp.int32)
    ptx = (part_of_speeches.reshape(N) * N_TYPES + types.reshape(N)).astype(
        jnp.int32)
    ptf = jnp.concatenate(
        [
            jnp.repeat(pos_table, N_TYPES, axis=0),
            jnp.tile(type_table, (N_POS, 1)),
            jnp.zeros((N_POS * N_TYPES, D_W - D_PT), jnp.float32),
        ],
        axis=1,
    )
    out = _sc_kernel(tok, ptx, word_table, ptf)
    return _tc_reshape(out)


# speedup vs baseline: 1.4092x; 1.4092x over previous
import functools

import jax
import jax.numpy as jnp
from jax import lax
from jax.experimental import pallas as pl
from jax.experimental.pallas import tpu as pltpu
from jax.experimental.pallas import tpu_sc as plsc

B, L = 1024, 200
N = B * L
D_W, D_P, D_T = 128, 32, 16
D_PT = D_P + D_T
D_OUT = D_W + D_PT
N_POS, N_TYPES = 50, 30

NC, NS = 2, 16
NW = NC * NS
B_PER_W = B // NW
CHUNK = L
G0, G1 = 128, CHUNK - 128


def _make_sc_kernel():
    mesh = plsc.VectorSubcoreMesh(core_axis_name="c", subcore_axis_name="s")

    @functools.partial(
        pl.kernel,
        mesh=mesh,
        out_type=jax.ShapeDtypeStruct((B, L, D_OUT), jnp.float32),
        scratch_types=[
            pltpu.VMEM((CHUNK,), jnp.int32),
            pltpu.VMEM((CHUNK,), jnp.int32),
            pltpu.VMEM((CHUNK,), jnp.int32),
            pltpu.VMEM((CHUNK,), jnp.int32),
            pltpu.VMEM((CHUNK, D_W), jnp.float32),
            pltpu.VMEM((1, CHUNK, D_OUT), jnp.float32),
            pltpu.VMEM((1, CHUNK, D_OUT), jnp.float32),
            pltpu.SemaphoreType.DMA,
            pltpu.SemaphoreType.DMA,
            pltpu.SemaphoreType.DMA,
            pltpu.SemaphoreType.DMA,
        ],
    )
    def k(tok_hbm, pt_hbm, wt_hbm, ptf_hbm, out_hbm,
          tok_i0, tok_i1, pt_i0, pt_i1, buf_pt, comb0, comb1,
          g_sem0, g_sem1, i_sem0, i_sem1):
        wid = lax.axis_index("s") * NC + lax.axis_index("c")
        b0 = wid * B_PER_W

        set0 = (tok_i0, pt_i0, comb0, g_sem0, i_sem0)
        set1 = (tok_i1, pt_i1, comb1, g_sem1, i_sem1)

        def start_idx(c, s):
            tok_i, pt_i, _, _, i_sem = s
            base = (b0 + c) * CHUNK
            pltpu.async_copy(tok_hbm.at[pl.ds(base, CHUNK)], tok_i, i_sem)
            pltpu.async_copy(pt_hbm.at[pl.ds(base, CHUNK)], pt_i, i_sem)

        def wait_idx(s):
            tok_i, pt_i, _, _, i_sem = s
            pltpu.make_async_copy(
                tok_hbm.at[pl.ds(0, CHUNK)], tok_i, i_sem).wait()
            pltpu.make_async_copy(
                pt_hbm.at[pl.ds(0, CHUNK)], pt_i, i_sem).wait()

        def issue_word(s):
            tok_i, _, comb, g_sem, _ = s
            pltpu.async_copy(
                wt_hbm.at[tok_i.at[pl.ds(0, G0)]],
                comb.at[0, pl.ds(0, G0), pl.ds(0, D_W)], g_sem)
            pltpu.async_copy(
                wt_hbm.at[tok_i.at[pl.ds(G0, G1)]],
                comb.at[0, pl.ds(G0, G1), pl.ds(0, D_W)], g_sem)

        def issue_ptf(s):
            _, pt_i, _, g_sem, _ = s
            pltpu.async_copy(
                ptf_hbm.at[pt_i.at[pl.ds(0, G0)]],
                buf_pt.at[pl.ds(0, G0)], g_sem)
            pltpu.async_copy(
                ptf_hbm.at[pt_i.at[pl.ds(G0, G1)]],
                buf_pt.at[pl.ds(G0, G1)], g_sem)

        def wait_gathers(s):
            _, _, comb, g_sem, _ = s
            pltpu.make_async_copy(
                wt_hbm.at[pl.ds(0, G0)],
                comb.at[0, pl.ds(0, G0), pl.ds(0, D_W)], g_sem).wait()
            pltpu.make_async_copy(
                wt_hbm.at[pl.ds(0, G1)],
                comb.at[0, pl.ds(G0, G1), pl.ds(0, D_W)], g_sem).wait()
            pltpu.make_async_copy(
                ptf_hbm.at[pl.ds(0, G0)],
                buf_pt.at[pl.ds(0, G0)], g_sem).wait()
            pltpu.make_async_copy(
                ptf_hbm.at[pl.ds(0, G1)],
                buf_pt.at[pl.ds(G0, G1)], g_sem).wait()

        def phase(c, cur, nxt, do_prefetch_idx, do_issue_next):
            _, _, comb, _, _ = cur

            if do_issue_next:
                wait_idx(nxt)
                issue_word(nxt)
            wait_gathers(cur)

            def stitch(r, carry):
                row_src = buf_pt.at[r]
                row_dst = comb.at[0, r]
                row_dst[pl.ds(D_W, 16)] = row_src[pl.ds(0, 16)]
                row_dst[pl.ds(D_W + 16, 16)] = row_src[pl.ds(16, 16)]
                row_dst[pl.ds(D_W + 32, 16)] = row_src[pl.ds(32, 16)]
                return carry

            lax.fori_loop(0, CHUNK, stitch, 0)
            if do_issue_next:
                issue_ptf(nxt)
            pltpu.sync_copy(comb, out_hbm.at[pl.ds(b0 + c, 1)])
            if do_prefetch_idx:
                start_idx(c + 2, cur)

        start_idx(0, set0)
        start_idx(1, set1)
        wait_idx(set0)
        issue_word(set0)
        issue_ptf(set0)

        def body(j, carry):
            c = 2 * j
            phase(c, set0, set1, True, True)
            phase(c + 1, set1, set0, True, True)
            return carry

        lax.fori_loop(0, (B_PER_W - 2) // 2, body, 0)
        phase(B_PER_W - 2, set0, set1, False, True)
        phase(B_PER_W - 1, set1, set0, False, False)

    return k


_sc_kernel = _make_sc_kernel()


def kernel(tokens, part_of_speeches, types, word_table, pos_table, type_table):
    tok = tokens.reshape(N).astype(jnp.int32)
    ptx = (part_of_speeches.reshape(N) * N_TYPES + types.reshape(N)).astype(
        jnp.int32)
    ptf = jnp.concatenate(
        [
            jnp.repeat(pos_table, N_TYPES, axis=0),
            jnp.tile(type_table, (N_POS, 1)),
            jnp.zeros((N_POS * N_TYPES, D_W - D_PT), jnp.float32),
        ],
        axis=1,
    )
    return _sc_kernel(tok, ptx, word_table, ptf)


# speedup vs baseline: 1.6538x; 1.1736x over previous
import functools

import jax
import jax.numpy as jnp
from jax import lax
from jax.experimental import pallas as pl
from jax.experimental.pallas import tpu as pltpu
from jax.experimental.pallas import tpu_sc as plsc

B, L = 1024, 200
N = B * L
D_W, D_P, D_T = 128, 32, 16
D_PT = D_P + D_T
D_OUT = D_W + D_PT
N_POS, N_TYPES = 50, 30

NC, NS = 2, 16
NW = NC * NS
ROWS_PER_W = N // NW
CHUNK = 128
N_CHUNKS = ROWS_PER_W // CHUNK


def _make_sc_kernel():
    mesh = plsc.VectorSubcoreMesh(core_axis_name="c", subcore_axis_name="s")

    @functools.partial(
        pl.kernel,
        mesh=mesh,
        out_type=jax.ShapeDtypeStruct((N, D_OUT), jnp.float32),
        scratch_types=[
            pltpu.VMEM((CHUNK,), jnp.int32),
            pltpu.VMEM((CHUNK,), jnp.int32),
            pltpu.VMEM((CHUNK,), jnp.int32),
            pltpu.VMEM((CHUNK,), jnp.int32),
            pltpu.VMEM((CHUNK, D_W), jnp.float32),
            pltpu.VMEM((CHUNK, D_W), jnp.float32),
            pltpu.VMEM((CHUNK, D_OUT), jnp.float32),
            pltpu.VMEM((CHUNK, D_OUT), jnp.float32),
            pltpu.SemaphoreType.DMA,
            pltpu.SemaphoreType.DMA,
            pltpu.SemaphoreType.DMA,
            pltpu.SemaphoreType.DMA,
            pltpu.SemaphoreType.DMA,
            pltpu.SemaphoreType.DMA,
        ],
    )
    def k(tok_hbm, pt_hbm, wt_hbm, ptf_hbm, out_hbm,
          tok_i0, tok_i1, pt_i0, pt_i1, bpt0, bpt1, comb0, comb1,
          g_sem0, g_sem1, i_sem0, i_sem1, w_sem0, w_sem1):
        wid = lax.axis_index("s") * NC + lax.axis_index("c")
        row0 = wid * ROWS_PER_W

        set0 = (tok_i0, pt_i0, bpt0, comb0, g_sem0, i_sem0, w_sem0)
        set1 = (tok_i1, pt_i1, bpt1, comb1, g_sem1, i_sem1, w_sem1)

        def start_idx(c, s):
            tok_i, pt_i, _, _, _, i_sem, _ = s
            b = row0 + c * CHUNK
            pltpu.async_copy(tok_hbm.at[pl.ds(b, CHUNK)], tok_i, i_sem)
            pltpu.async_copy(pt_hbm.at[pl.ds(b, CHUNK)], pt_i, i_sem)

        def wait_idx(s):
            tok_i, pt_i, _, _, _, i_sem, _ = s
            pltpu.make_async_copy(
                tok_hbm.at[pl.ds(0, CHUNK)], tok_i, i_sem).wait()
            pltpu.make_async_copy(
                pt_hbm.at[pl.ds(0, CHUNK)], pt_i, i_sem).wait()

        def issue_gathers(s):
            tok_i, pt_i, bpt, comb, g_sem, _, _ = s
            pltpu.async_copy(
                wt_hbm.at[tok_i], comb.at[:, pl.ds(0, D_W)], g_sem)
            pltpu.async_copy(ptf_hbm.at[pt_i], bpt, g_sem)

        def wait_gathers(s):
            _, _, bpt, comb, g_sem, _, _ = s
            pltpu.make_async_copy(
                wt_hbm.at[pl.ds(0, CHUNK)],
                comb.at[:, pl.ds(0, D_W)], g_sem).wait()
            pltpu.make_async_copy(
                ptf_hbm.at[pl.ds(0, CHUNK)], bpt, g_sem).wait()

        def start_write(c, s):
            _, _, _, comb, _, _, w_sem = s
            pltpu.async_copy(
                comb, out_hbm.at[pl.ds(row0 + c * CHUNK, CHUNK)], w_sem)

        def wait_write(s):
            _, _, _, comb, _, _, w_sem = s
            pltpu.make_async_copy(
                comb, out_hbm.at[pl.ds(0, CHUNK)], w_sem).wait()

        def phase(c, cur, nxt, do_prefetch_idx, do_issue_next, do_wait_write):
            _, _, bpt, comb, _, _, _ = cur
            if do_issue_next:
                wait_idx(nxt)
                if do_wait_write:
                    wait_write(nxt)
                issue_gathers(nxt)
            wait_gathers(cur)

            def stitch(r, carry):
                row_src = bpt.at[r]
                row_dst = comb.at[r]
                row_dst[pl.ds(D_W, 16)] = row_src[pl.ds(0, 16)]
                row_dst[pl.ds(D_W + 16, 16)] = row_src[pl.ds(16, 16)]
                row_dst[pl.ds(D_W + 32, 16)] = row_src[pl.ds(32, 16)]
                return carry

            lax.fori_loop(0, CHUNK, stitch, 0)
            start_write(c, cur)
            if do_prefetch_idx:
                start_idx(c + 2, cur)

        start_idx(0, set0)
        start_idx(1, set1)
        wait_idx(set0)
        issue_gathers(set0)

        phase(0, set0, set1, True, True, False)

        def body(j, carry):
            c = 2 * j + 1
            phase(c, set1, set0, True, True, True)
            phase(c + 1, set0, set1, True, True, True)
            return carry

        lax.fori_loop(0, (N_CHUNKS - 4) // 2, body, 0)
        phase(N_CHUNKS - 3, set1, set0, True, True, True)
        phase(N_CHUNKS - 2, set0, set1, False, True, True)
        phase(N_CHUNKS - 1, set1, set0, False, False, False)
        wait_write(set0)
        wait_write(set1)

    return k


_sc_kernel = _make_sc_kernel()


def kernel(tokens, part_of_speeches, types, word_table, pos_table, type_table):
    tok = tokens.reshape(N).astype(jnp.int32)
    ptx = (part_of_speeches.reshape(N) * N_TYPES + types.reshape(N)).astype(
        jnp.int32)
    ptf = jnp.concatenate(
        [
            jnp.repeat(pos_table, N_TYPES, axis=0),
            jnp.tile(type_table, (N_POS, 1)),
            jnp.zeros((N_POS * N_TYPES, D_W - D_PT), jnp.float32),
        ],
        axis=1,
    )
    out = _sc_kernel(tok, ptx, word_table, ptf)
    return out.reshape(B, L, D_OUT)


# speedup vs baseline: 1.6684x; 1.0088x over previous
import functools

import jax
import jax.numpy as jnp
from jax import lax
from jax.experimental import pallas as pl
from jax.experimental.pallas import tpu as pltpu
from jax.experimental.pallas import tpu_sc as plsc

B, L = 1024, 200
N = B * L
D_W, D_P, D_T = 128, 32, 16
D_PT = D_P + D_T
D_OUT = D_W + D_PT
N_POS, N_TYPES = 50, 30

NC, NS = 2, 16
NW = NC * NS
ROWS_PER_W = N // NW
CHUNK = 128
N_CHUNKS = ROWS_PER_W // CHUNK


def _make_sc_kernel():
    mesh = plsc.VectorSubcoreMesh(core_axis_name="c", subcore_axis_name="s")

    @functools.partial(
        pl.kernel,
        mesh=mesh,
        out_type=jax.ShapeDtypeStruct((N, D_OUT), jnp.float32),
        scratch_types=[
            pltpu.VMEM((CHUNK,), jnp.int32),
            pltpu.VMEM((CHUNK,), jnp.int32),
            pltpu.VMEM((CHUNK,), jnp.int32),
            pltpu.VMEM((CHUNK,), jnp.int32),
            pltpu.VMEM((CHUNK, D_W), jnp.float32),
            pltpu.VMEM((CHUNK, D_W), jnp.float32),
            pltpu.VMEM((CHUNK, D_OUT), jnp.float32),
            pltpu.VMEM((CHUNK, D_OUT), jnp.float32),
            pltpu.SemaphoreType.DMA,
            pltpu.SemaphoreType.DMA,
            pltpu.SemaphoreType.DMA,
            pltpu.SemaphoreType.DMA,
            pltpu.SemaphoreType.DMA,
            pltpu.SemaphoreType.DMA,
        ],
    )
    def k(tok_hbm, pt_hbm, wt_hbm, ptf_hbm, out_hbm,
          tok_i0, tok_i1, pt_i0, pt_i1, bpt0, bpt1, comb0, comb1,
          g_sem0, g_sem1, i_sem0, i_sem1, w_sem0, w_sem1):
        wid = lax.axis_index("s") * NC + lax.axis_index("c")
        row0 = wid * ROWS_PER_W

        set0 = (tok_i0, pt_i0, bpt0, comb0, g_sem0, i_sem0, w_sem0)
        set1 = (tok_i1, pt_i1, bpt1, comb1, g_sem1, i_sem1, w_sem1)

        def start_idx(c, s):
            tok_i, pt_i, _, _, _, i_sem, _ = s
            b = row0 + c * CHUNK
            pltpu.async_copy(tok_hbm.at[pl.ds(b, CHUNK)], tok_i, i_sem)
            pltpu.async_copy(pt_hbm.at[pl.ds(b, CHUNK)], pt_i, i_sem)

        def wait_idx(s):
            tok_i, pt_i, _, _, _, i_sem, _ = s
            pltpu.make_async_copy(
                tok_hbm.at[pl.ds(0, CHUNK)], tok_i, i_sem).wait()
            pltpu.make_async_copy(
                pt_hbm.at[pl.ds(0, CHUNK)], pt_i, i_sem).wait()

        def issue_gathers(s):
            tok_i, pt_i, bpt, comb, g_sem, _, _ = s
            pltpu.async_copy(
                wt_hbm.at[tok_i], comb.at[:, pl.ds(0, D_W)], g_sem)
            pltpu.async_copy(ptf_hbm.at[pt_i], bpt, g_sem)

        def wait_gathers(s):
            _, _, bpt, comb, g_sem, _, _ = s
            pltpu.make_async_copy(
                wt_hbm.at[pl.ds(0, CHUNK)],
                comb.at[:, pl.ds(0, D_W)], g_sem).wait()
            pltpu.make_async_copy(
                ptf_hbm.at[pl.ds(0, CHUNK)], bpt, g_sem).wait()

        def start_write(c, s):
            _, _, _, comb, _, _, w_sem = s
            pltpu.async_copy(
                comb, out_hbm.at[pl.ds(row0 + c * CHUNK, CHUNK)], w_sem)

        def wait_write(s):
            _, _, _, comb, _, _, w_sem = s
            pltpu.make_async_copy(
                comb, out_hbm.at[pl.ds(0, CHUNK)], w_sem).wait()

        def phase(c, cur, nxt, do_prefetch_idx, do_issue_next, do_wait_write):
            _, _, bpt, comb, _, _, _ = cur
            if do_issue_next:
                wait_idx(nxt)
                if do_wait_write:
                    wait_write(nxt)
                issue_gathers(nxt)
            wait_gathers(cur)

            @functools.partial(plsc.parallel_loop, 0, CHUNK, unroll=8)
            def stitch(r):
                row_src = bpt.at[r]
                row_dst = comb.at[r]
                row_dst[pl.ds(D_W, 16)] = row_src[pl.ds(0, 16)]
                row_dst[pl.ds(D_W + 16, 16)] = row_src[pl.ds(16, 16)]
                row_dst[pl.ds(D_W + 32, 16)] = row_src[pl.ds(32, 16)]
            start_write(c, cur)
            if do_prefetch_idx:
                start_idx(c + 2, cur)

        start_idx(0, set0)
        start_idx(1, set1)
        wait_idx(set0)
        issue_gathers(set0)

        phase(0, set0, set1, True, True, False)

        def body(j, carry):
            c = 2 * j + 1
            phase(c, set1, set0, True, True, True)
            phase(c + 1, set0, set1, True, True, True)
            return carry

        lax.fori_loop(0, (N_CHUNKS - 4) // 2, body, 0)
        phase(N_CHUNKS - 3, set1, set0, True, True, True)
        phase(N_CHUNKS - 2, set0, set1, False, True, True)
        phase(N_CHUNKS - 1, set1, set0, False, False, False)
        wait_write(set0)
        wait_write(set1)

    return k


_sc_kernel = _make_sc_kernel()


def kernel(tokens, part_of_speeches, types, word_table, pos_table, type_table):
    tok = tokens.reshape(N).astype(jnp.int32)
    ptx = (part_of_speeches.reshape(N) * N_TYPES + types.reshape(N)).astype(
        jnp.int32)
    ptf = jnp.concatenate(
        [
            jnp.repeat(pos_table, N_TYPES, axis=0),
            jnp.tile(type_table, (N_POS, 1)),
            jnp.zeros((N_POS * N_TYPES, D_W - D_PT), jnp.float32),
        ],
        axis=1,
    )
    out = _sc_kernel(tok, ptx, word_table, ptf)
    return out.reshape(B, L, D_OUT)
